# baseline (speedup 1.0000x reference)
import functools

import jax
import jax.numpy as jnp
import numpy as np
from jax import lax
from jax.experimental import pallas as pl
from jax.experimental.pallas import tpu as pltpu
from jax.experimental.pallas import tpu_sc as plsc


def _top3_body(xyz1_ref, xyz2_ref, gidx_ref, wrep_ref, *, tn, s):
    b = pl.program_id(0)
    q = xyz1_ref[0]
    k = xyz2_ref[0]
    q0, q1, q2 = q[0], q[1], q[2]
    k0, k1, k2 = k[0], k[1], k[2]
    sq1 = q0 * q0 + q1 * q1 + q2 * q2
    sq2 = k0 * k0 + k1 * k1 + k2 * k2
    qk = lax.dot_general(q.astype(jnp.bfloat16), k.astype(jnp.bfloat16),
                         (((0,), (0,)), ((), ())),
                         preferred_element_type=jnp.float32)
    d = sq1[:, None] + sq2[None, :] - 2.0 * qk

    iota = lax.broadcasted_iota(jnp.int32, (tn, s), 1)
    inf = jnp.float32(np.inf)
    dcur = d
    recips, idxs = [], []
    for _ in range(3):
        mv = jnp.min(dcur, axis=1)
        eq = dcur == mv[:, None]
        ii = jnp.min(jnp.where(eq, iota, s), axis=1)
        sel = iota == ii[:, None]
        recips.append(1.0 / (mv + 1e-8))
        idxs.append(ii)
        dcur = jnp.where(sel, inf, dcur)
    norm = recips[0] + recips[1] + recips[2]
    for j in range(3):
        gidx_ref[j, 0, 0] = idxs[j] + b * s
        wj = recips[j] / norm
        wrep_ref[j, 0] = jnp.broadcast_to(wj[:, None], (tn, 16))


def _sc_gather_body(table_hbm, gidx_hbm, wrep_hbm, out_hbm,
                    idx_v, w_buf, rows_v, acc_v,
                    gsem0, gsem1, ssem0, ssem1, *, qpw, chunk, c2, nc):
    wid = lax.axis_index("s") * nc + lax.axis_index("c")
    base = wid * qpw
    nch = qpw // chunk
    cblocks = c2 // 16
    gsems = (gsem0, gsem1)
    ssems = (ssem0, ssem1)

    pltpu.sync_copy(gidx_hbm.at[pl.ds(3 * base, 3 * qpw)], idx_v)

    def fire(ci, p):
        off3 = ci * (3 * chunk)
        pltpu.async_copy(
            table_hbm.at[idx_v.at[pl.ds(off3, 3 * chunk)]],
            rows_v.at[p], gsems[p])
        pltpu.async_copy(wrep_hbm.at[pl.ds(3 * base + off3, 3 * chunk)],
                         w_buf.at[p], gsems[p])

    def wait_gather(p):
        pltpu.make_async_copy(table_hbm.at[pl.ds(0, 3 * chunk)],
                              rows_v.at[p], gsems[p]).wait()
        pltpu.make_async_copy(wrep_hbm.at[pl.ds(0, 3 * chunk)],
                              w_buf.at[p], gsems[p]).wait()

    def compute(ci, p):
        for i in range(chunk):
            w0 = w_buf[p, i]
            w1 = w_buf[p, chunk + i]
            w2 = w_buf[p, 2 * chunk + i]
            for cb in range(cblocks):
                r0 = rows_v[p, i, pl.ds(cb * 16, 16)]
                r1 = rows_v[p, chunk + i, pl.ds(cb * 16, 16)]
                r2 = rows_v[p, 2 * chunk + i, pl.ds(cb * 16, 16)]
                acc_v[p, i, pl.ds(cb * 16, 16)] = r0 * w0 + r1 * w1 + r2 * w2
        pltpu.async_copy(acc_v.at[p],
                         out_hbm.at[pl.ds(base + ci * chunk, chunk)], ssems[p])

    def wait_store(p):
        pltpu.make_async_copy(acc_v.at[p],
                              out_hbm.at[pl.ds(base, chunk)], ssems[p]).wait()

    fire(0, 0)
    fire(1, 1)

    def pair_body(it, carry):
        for p in range(2):
            ci = 2 * it + p
            wait_gather(p)

            @pl.when(it >= 1)
            def _():
                wait_store(p)

            compute(ci, p)
            fire(jnp.minimum(ci + 2, nch - 1), p)
        return carry

    lax.fori_loop(0, nch // 2, pair_body, 0)
    for p in range(2):
        wait_gather(p)
        wait_store(p)


def _mlp_body(p1_ref, it_ref, w1a_ref, w1b_ref, b1_ref, w2_ref, b2_ref,
              out_ref):
    bf = jnp.bfloat16
    p1 = p1_ref[0]
    it = it_ref[0]
    h = lax.dot_general(w1a_ref[...].astype(bf), p1.astype(bf),
                        (((1,), (0,)), ((), ())),
                        preferred_element_type=jnp.float32)
    h = h + lax.dot_general(w1b_ref[...].astype(bf), it.astype(bf),
                            (((1,), (1,)), ((), ())),
                            preferred_element_type=jnp.float32)
    h = jnp.maximum(h + b1_ref[...][:, 0][:, None], 0.0)
    h2 = lax.dot_general(w2_ref[...].astype(bf), h.astype(bf),
                         (((1,), (0,)), ((), ())),
                         preferred_element_type=jnp.float32)
    h2 = jnp.maximum(h2 + b2_ref[...][:, 0][:, None], 0.0)
    out_ref[0] = h2


def kernel(xyz1, xyz2, points1, points2, W1, b1, W2, b2):
    B, _, N = xyz1.shape
    S = xyz2.shape[2]
    C1 = points1.shape[1]
    C2 = points2.shape[1]
    O1 = W1.shape[0]
    O2 = W2.shape[0]
    TN = min(512, N)
    NW = 32
    QPW = (B * N) // NW
    CHUNK = 16

    top3 = pl.pallas_call(
        functools.partial(_top3_body, tn=TN, s=S),
        grid=(B, N // TN),
        in_specs=[
            pl.BlockSpec((1, 3, TN), lambda b, n: (b, 0, n)),
            pl.BlockSpec((1, 3, S), lambda b, n: (b, 0, 0)),
        ],
        out_specs=[
            pl.BlockSpec((3, 1, 1, TN), lambda b, n: (0, b, 0, n)),
            pl.BlockSpec((3, 1, TN, 16), lambda b, n: (0, b, n, 0)),
        ],
        out_shape=[
            jax.ShapeDtypeStruct((3, B, 1, N), jnp.int32),
            jax.ShapeDtypeStruct((3, B, N, 16), jnp.float32),
        ],
        compiler_params=pltpu.CompilerParams(
            dimension_semantics=("parallel", "arbitrary"),
        ),
    )(xyz1, xyz2)
    nchunks = (B * N) // CHUNK
    gidx = (top3[0].reshape(3, nchunks, CHUNK)
            .transpose(1, 0, 2).reshape(3 * B * N))
    wrep = (top3[1].reshape(3, nchunks, CHUNK, 16)
            .transpose(1, 0, 2, 3).reshape(3 * B * N, 16))

    table = jnp.transpose(points2, (0, 2, 1)).reshape(B * S, C2)
    mesh = plsc.VectorSubcoreMesh(core_axis_name="c", subcore_axis_name="s",
                                  num_cores=2, num_subcores=16)
    interp = pl.kernel(
        functools.partial(_sc_gather_body, qpw=QPW, chunk=CHUNK, c2=C2, nc=2),
        out_type=jax.ShapeDtypeStruct((B * N, C2), jnp.float32),
        mesh=mesh,
        scratch_types=[
            pltpu.VMEM((3 * QPW,), jnp.int32),
            pltpu.VMEM((2, 3 * CHUNK, 16), jnp.float32),
            pltpu.VMEM((2, 3 * CHUNK, C2), jnp.float32),
            pltpu.VMEM((2, CHUNK, C2), jnp.float32),
            pltpu.SemaphoreType.DMA,
            pltpu.SemaphoreType.DMA,
            pltpu.SemaphoreType.DMA,
            pltpu.SemaphoreType.DMA,
        ],
    )(table, gidx, wrep)
    interp = interp.reshape(B, N, C2)

    w1a = W1[:, :C1]
    w1b = W1[:, C1:]
    b1c = b1[:, None]
    b2c = b2[:, None]
    out = pl.pallas_call(
        _mlp_body,
        grid=(B, N // TN),
        in_specs=[
            pl.BlockSpec((1, C1, TN), lambda b, n: (b, 0, n)),
            pl.BlockSpec((1, TN, C2), lambda b, n: (b, n, 0)),
            pl.BlockSpec((O1, C1), lambda b, n: (0, 0)),
            pl.BlockSpec((O1, C2), lambda b, n: (0, 0)),
            pl.BlockSpec((O1, 1), lambda b, n: (0, 0)),
            pl.BlockSpec((O2, O1), lambda b, n: (0, 0)),
            pl.BlockSpec((O2, 1), lambda b, n: (0, 0)),
        ],
        out_specs=pl.BlockSpec((1, O2, TN), lambda b, n: (b, 0, n)),
        out_shape=jax.ShapeDtypeStruct((B, O2, N), jnp.float32),
        compiler_params=pltpu.CompilerParams(
            dimension_semantics=("parallel", "arbitrary"),
        ),
    )(points1, interp, w1a, w1b, b1c, W2, b2c)
    return out


# speedup vs baseline: 1.1336x; 1.1336x over previous
import functools

import jax
import jax.numpy as jnp
import numpy as np
from jax import lax
from jax.experimental import pallas as pl
from jax.experimental.pallas import tpu as pltpu
from jax.experimental.pallas import tpu_sc as plsc


def _top3_body(xyz1_ref, xyz2_ref, gidx_ref, wrep_ref, *, tn, s):
    b = pl.program_id(0)
    q = xyz1_ref[0]
    k = xyz2_ref[0]
    q0, q1, q2 = q[0], q[1], q[2]
    k0, k1, k2 = k[0], k[1], k[2]
    sq1 = q0 * q0 + q1 * q1 + q2 * q2
    sq2 = k0 * k0 + k1 * k1 + k2 * k2
    qk = lax.dot_general(q.astype(jnp.bfloat16), k.astype(jnp.bfloat16),
                         (((0,), (0,)), ((), ())),
                         preferred_element_type=jnp.float32)
    d = sq1[:, None] + sq2[None, :] - 2.0 * qk

    iota = lax.broadcasted_iota(jnp.int32, (tn, s), 1)
    inf = jnp.float32(np.inf)
    dcur = d
    recips, idxs = [], []
    for _ in range(3):
        mv = jnp.min(dcur, axis=1)
        eq = dcur == mv[:, None]
        ii = jnp.min(jnp.where(eq, iota, s), axis=1)
        sel = iota == ii[:, None]
        recips.append(1.0 / (mv + 1e-8))
        idxs.append(ii)
        dcur = jnp.where(sel, inf, dcur)
    norm = recips[0] + recips[1] + recips[2]
    for j in range(3):
        gidx_ref[j, 0, 0] = idxs[j] + b * s
        wj = recips[j] / norm
        wrep_ref[j, 0] = jnp.broadcast_to(wj[:, None], (tn, 16))


def _sc_gather_body(table_hbm, gidx_hbm, wrep_hbm, out_hbm,
                    idx_v, w_buf, rows_v, acc_v,
                    gsem0, gsem1, ssem0, ssem1, *, qpw, chunk, c2, nc):
    wid = lax.axis_index("s") * nc + lax.axis_index("c")
    base = wid * qpw
    nch = qpw // chunk
    cblocks = c2 // 16
    gsems = (gsem0, gsem1)
    ssems = (ssem0, ssem1)

    pltpu.sync_copy(gidx_hbm.at[pl.ds(3 * base, 3 * qpw)], idx_v)

    def fire(ci, p):
        off3 = ci * (3 * chunk)
        pltpu.async_copy(
            table_hbm.at[idx_v.at[pl.ds(off3, 3 * chunk)]],
            rows_v.at[p], gsems[p])
        pltpu.async_copy(wrep_hbm.at[pl.ds(3 * base + off3, 3 * chunk)],
                         w_buf.at[p], gsems[p])

    def wait_gather(p):
        pltpu.make_async_copy(table_hbm.at[pl.ds(0, 3 * chunk)],
                              rows_v.at[p], gsems[p]).wait()
        pltpu.make_async_copy(wrep_hbm.at[pl.ds(0, 3 * chunk)],
                              w_buf.at[p], gsems[p]).wait()

    def compute(ci, p):
        for i in range(chunk):
            w0 = w_buf[p, i]
            w1 = w_buf[p, chunk + i]
            w2 = w_buf[p, 2 * chunk + i]
            for cb in range(cblocks):
                r0 = rows_v[p, i, pl.ds(cb * 16, 16)]
                r1 = rows_v[p, chunk + i, pl.ds(cb * 16, 16)]
                r2 = rows_v[p, 2 * chunk + i, pl.ds(cb * 16, 16)]
                acc_v[p, i, pl.ds(cb * 16, 16)] = r0 * w0 + r1 * w1 + r2 * w2
        pltpu.async_copy(acc_v.at[p],
                         out_hbm.at[pl.ds(base + ci * chunk, chunk)], ssems[p])

    def wait_store(p):
        pltpu.make_async_copy(acc_v.at[p],
                              out_hbm.at[pl.ds(base, chunk)], ssems[p]).wait()

    fire(0, 0)
    fire(1, 1)

    def pair_body(it, carry):
        for p in range(2):
            ci = 2 * it + p
            wait_gather(p)

            @pl.when(it >= 1)
            def _():
                wait_store(p)

            compute(ci, p)
            fire(jnp.minimum(ci + 2, nch - 1), p)
        return carry

    lax.fori_loop(0, nch // 2, pair_body, 0)
    for p in range(2):
        wait_gather(p)
        wait_store(p)


def _mlp_body(p1_ref, it_ref, w1a_ref, w1b_ref, b1_ref, w2_ref, b2_ref,
              out_ref):
    bf = jnp.bfloat16
    p1 = p1_ref[0]
    it = it_ref[0]
    h = lax.dot_general(w1a_ref[...].astype(bf), p1.astype(bf),
                        (((1,), (0,)), ((), ())),
                        preferred_element_type=jnp.float32)
    h = h + lax.dot_general(w1b_ref[...].astype(bf), it.astype(bf),
                            (((1,), (1,)), ((), ())),
                            preferred_element_type=jnp.float32)
    h = jnp.maximum(h + b1_ref[...][:, 0][:, None], 0.0)
    h2 = lax.dot_general(w2_ref[...].astype(bf), h.astype(bf),
                         (((1,), (0,)), ((), ())),
                         preferred_element_type=jnp.float32)
    h2 = jnp.maximum(h2 + b2_ref[...][:, 0][:, None], 0.0)
    out_ref[0] = h2


def kernel(xyz1, xyz2, points1, points2, W1, b1, W2, b2):
    B, _, N = xyz1.shape
    S = xyz2.shape[2]
    C1 = points1.shape[1]
    C2 = points2.shape[1]
    O1 = W1.shape[0]
    O2 = W2.shape[0]
    TN = min(512, N)
    NW = 32
    CHUNK = 16

    w1a = W1[:, :C1]
    w1b = W1[:, C1:]
    b1c = b1[:, None]
    b2c = b2[:, None]

    def top3_stage(x1, x2):
        b = x1.shape[0]
        return pl.pallas_call(
            functools.partial(_top3_body, tn=TN, s=S),
            grid=(b, N // TN),
            in_specs=[
                pl.BlockSpec((1, 3, TN), lambda b, n: (b, 0, n)),
                pl.BlockSpec((1, 3, S), lambda b, n: (b, 0, 0)),
            ],
            out_specs=[
                pl.BlockSpec((3, 1, 1, TN), lambda b, n: (0, b, 0, n)),
                pl.BlockSpec((3, 1, TN, 16), lambda b, n: (0, b, n, 0)),
            ],
            out_shape=[
                jax.ShapeDtypeStruct((3, b, 1, N), jnp.int32),
                jax.ShapeDtypeStruct((3, b, N, 16), jnp.float32),
            ],
            compiler_params=pltpu.CompilerParams(
                dimension_semantics=("parallel", "arbitrary"),
            ),
        )(x1, x2)

    def sc_stage(table, gidx4, wrep4):
        b = gidx4.shape[1]
        qpw = (b * N) // NW
        nchunks = (b * N) // CHUNK
        gidx = (gidx4.reshape(3, nchunks, CHUNK)
                .transpose(1, 0, 2).reshape(3 * b * N))
        wrep = (wrep4.reshape(3, nchunks, CHUNK, 16)
                .transpose(1, 0, 2, 3).reshape(3 * b * N, 16))
        mesh = plsc.VectorSubcoreMesh(core_axis_name="c", subcore_axis_name="s",
                                      num_cores=2, num_subcores=16)
        interp = pl.kernel(
            functools.partial(_sc_gather_body, qpw=qpw, chunk=CHUNK, c2=C2,
                              nc=2),
            out_type=jax.ShapeDtypeStruct((b * N, C2), jnp.float32),
            mesh=mesh,
            scratch_types=[
                pltpu.VMEM((3 * qpw,), jnp.int32),
                pltpu.VMEM((2, 3 * CHUNK, 16), jnp.float32),
                pltpu.VMEM((2, 3 * CHUNK, C2), jnp.float32),
                pltpu.VMEM((2, CHUNK, C2), jnp.float32),
                pltpu.SemaphoreType.DMA,
                pltpu.SemaphoreType.DMA,
                pltpu.SemaphoreType.DMA,
                pltpu.SemaphoreType.DMA,
            ],
        )(table, gidx, wrep)
        return interp.reshape(b, N, C2)

    def mlp_stage(p1, interp):
        b = p1.shape[0]
        return pl.pallas_call(
            _mlp_body,
            grid=(b, N // TN),
            in_specs=[
                pl.BlockSpec((1, C1, TN), lambda b, n: (b, 0, n)),
                pl.BlockSpec((1, TN, C2), lambda b, n: (b, n, 0)),
                pl.BlockSpec((O1, C1), lambda b, n: (0, 0)),
                pl.BlockSpec((O1, C2), lambda b, n: (0, 0)),
                pl.BlockSpec((O1, 1), lambda b, n: (0, 0)),
                pl.BlockSpec((O2, O1), lambda b, n: (0, 0)),
                pl.BlockSpec((O2, 1), lambda b, n: (0, 0)),
            ],
            out_specs=pl.BlockSpec((1, O2, TN), lambda b, n: (b, 0, n)),
            out_shape=jax.ShapeDtypeStruct((b, O2, N), jnp.float32),
            compiler_params=pltpu.CompilerParams(
                dimension_semantics=("parallel", "arbitrary"),
            ),
        )(p1, interp, w1a, w1b, b1c, W2, b2c)

    half = B // 2
    outs = []
    tops = []
    for g in range(2):
        sl = slice(g * half, (g + 1) * half)
        tops.append((top3_stage(xyz1[sl], xyz2[sl]), sl))
    for g in range(2):
        (gidx4, wrep4), sl = tops[g]
        table = jnp.transpose(points2[sl], (0, 2, 1)).reshape(half * S, C2)
        interp = sc_stage(table, gidx4, wrep4)
        outs.append(mlp_stage(points1[sl], interp))
    return jnp.concatenate(outs, axis=0)


# speedup vs baseline: 1.2055x; 1.0635x over previous
import functools

import jax
import jax.numpy as jnp
import numpy as np
from jax import lax
from jax.experimental import pallas as pl
from jax.experimental.pallas import tpu as pltpu
from jax.experimental.pallas import tpu_sc as plsc


def _top3_body(xyz1_ref, xyz2_ref, gidx_ref, wrep_ref, *, tn, s):
    b = pl.program_id(0)
    q = xyz1_ref[0]
    k = xyz2_ref[0]
    q0, q1, q2 = q[0], q[1], q[2]
    k0, k1, k2 = k[0], k[1], k[2]
    sq1 = q0 * q0 + q1 * q1 + q2 * q2
    sq2 = k0 * k0 + k1 * k1 + k2 * k2
    qk = lax.dot_general(q.astype(jnp.bfloat16), k.astype(jnp.bfloat16),
                         (((0,), (0,)), ((), ())),
                         preferred_element_type=jnp.float32)
    d = sq1[:, None] + sq2[None, :] - 2.0 * qk

    iota = lax.broadcasted_iota(jnp.int32, (tn, s), 1)
    inf = jnp.float32(np.inf)
    dcur = d
    recips, idxs = [], []
    for _ in range(3):
        mv = jnp.min(dcur, axis=1)
        eq = dcur == mv[:, None]
        ii = jnp.min(jnp.where(eq, iota, s), axis=1)
        sel = iota == ii[:, None]
        recips.append(1.0 / (mv + 1e-8))
        idxs.append(ii)
        dcur = jnp.where(sel, inf, dcur)
    norm = recips[0] + recips[1] + recips[2]
    for j in range(3):
        gidx_ref[j, 0, 0] = idxs[j] + b * s
        wj = recips[j] / norm
        wrep_ref[j, 0] = jnp.broadcast_to(wj[:, None], (tn, 16))


def _sc_gather_body(table_hbm, gidx_hbm, wrep_hbm, out_hbm,
                    idx_v, w_buf, rows_v, acc_v,
                    gsem0, gsem1, ssem0, ssem1, *, qpw, chunk, c2, nc):
    wid = lax.axis_index("s") * nc + lax.axis_index("c")
    base = wid * qpw
    nch = qpw // chunk
    cblocks = c2 // 16
    gsems = (gsem0, gsem1)
    ssems = (ssem0, ssem1)

    pltpu.sync_copy(gidx_hbm.at[pl.ds(3 * base, 3 * qpw)], idx_v)

    def fire(ci, p):
        off3 = ci * (3 * chunk)
        pltpu.async_copy(
            table_hbm.at[idx_v.at[pl.ds(off3, 3 * chunk)]],
            rows_v.at[p], gsems[p])
        pltpu.async_copy(wrep_hbm.at[pl.ds(3 * base + off3, 3 * chunk)],
                         w_buf.at[p], gsems[p])

    def wait_gather(p):
        pltpu.make_async_copy(table_hbm.at[pl.ds(0, 3 * chunk)],
                              rows_v.at[p], gsems[p]).wait()
        pltpu.make_async_copy(wrep_hbm.at[pl.ds(0, 3 * chunk)],
                              w_buf.at[p], gsems[p]).wait()

    def compute(ci, p):
        for i in range(chunk):
            w0 = w_buf[p, i]
            w1 = w_buf[p, chunk + i]
            w2 = w_buf[p, 2 * chunk + i]
            for cb in range(cblocks):
                r0 = rows_v[p, i, pl.ds(cb * 16, 16)]
                r1 = rows_v[p, chunk + i, pl.ds(cb * 16, 16)]
                r2 = rows_v[p, 2 * chunk + i, pl.ds(cb * 16, 16)]
                acc_v[p, i, pl.ds(cb * 16, 16)] = r0 * w0 + r1 * w1 + r2 * w2
        pltpu.async_copy(acc_v.at[p],
                         out_hbm.at[pl.ds(base + ci * chunk, chunk)], ssems[p])

    def wait_store(p):
        pltpu.make_async_copy(acc_v.at[p],
                              out_hbm.at[pl.ds(base, chunk)], ssems[p]).wait()

    fire(0, 0)
    fire(1, 1)

    def pair_body(it, carry):
        for p in range(2):
            ci = 2 * it + p
            wait_gather(p)

            @pl.when(it >= 1)
            def _():
                wait_store(p)

            compute(ci, p)
            fire(jnp.minimum(ci + 2, nch - 1), p)
        return carry

    lax.fori_loop(0, nch // 2, pair_body, 0)
    for p in range(2):
        wait_gather(p)
        wait_store(p)


def _mlp_body(p1_ref, it_ref, w1a_ref, w1b_ref, b1_ref, w2_ref, b2_ref,
              out_ref):
    bf = jnp.bfloat16
    p1 = p1_ref[0]
    it = it_ref[0]
    h = lax.dot_general(w1a_ref[...].astype(bf), p1.astype(bf),
                        (((1,), (0,)), ((), ())),
                        preferred_element_type=jnp.float32)
    h = h + lax.dot_general(w1b_ref[...].astype(bf), it.astype(bf),
                            (((1,), (1,)), ((), ())),
                            preferred_element_type=jnp.float32)
    h = jnp.maximum(h + b1_ref[...][:, 0][:, None], 0.0)
    h2 = lax.dot_general(w2_ref[...].astype(bf), h.astype(bf),
                         (((1,), (0,)), ((), ())),
                         preferred_element_type=jnp.float32)
    h2 = jnp.maximum(h2 + b2_ref[...][:, 0][:, None], 0.0)
    out_ref[0] = h2


def kernel(xyz1, xyz2, points1, points2, W1, b1, W2, b2):
    B, _, N = xyz1.shape
    S = xyz2.shape[2]
    C1 = points1.shape[1]
    C2 = points2.shape[1]
    O1 = W1.shape[0]
    O2 = W2.shape[0]
    TN = min(512, N)
    NW = 32
    CHUNK = 16

    w1a = W1[:, :C1]
    w1b = W1[:, C1:]
    b1c = b1[:, None]
    b2c = b2[:, None]

    def top3_stage(x1, x2):
        b = x1.shape[0]
        return pl.pallas_call(
            functools.partial(_top3_body, tn=TN, s=S),
            grid=(b, N // TN),
            in_specs=[
                pl.BlockSpec((1, 3, TN), lambda b, n: (b, 0, n)),
                pl.BlockSpec((1, 3, S), lambda b, n: (b, 0, 0)),
            ],
            out_specs=[
                pl.BlockSpec((3, 1, 1, TN), lambda b, n: (0, b, 0, n)),
                pl.BlockSpec((3, 1, TN, 16), lambda b, n: (0, b, n, 0)),
            ],
            out_shape=[
                jax.ShapeDtypeStruct((3, b, 1, N), jnp.int32),
                jax.ShapeDtypeStruct((3, b, N, 16), jnp.float32),
            ],
            compiler_params=pltpu.CompilerParams(
                dimension_semantics=("parallel", "arbitrary"),
            ),
        )(x1, x2)

    def sc_stage(table, gidx4, wrep4):
        b = gidx4.shape[1]
        qpw = (b * N) // NW
        nchunks = (b * N) // CHUNK
        gidx = (gidx4.reshape(3, nchunks, CHUNK)
                .transpose(1, 0, 2).reshape(3 * b * N))
        wrep = (wrep4.reshape(3, nchunks, CHUNK, 16)
                .transpose(1, 0, 2, 3).reshape(3 * b * N, 16))
        mesh = plsc.VectorSubcoreMesh(core_axis_name="c", subcore_axis_name="s",
                                      num_cores=2, num_subcores=16)
        interp = pl.kernel(
            functools.partial(_sc_gather_body, qpw=qpw, chunk=CHUNK, c2=C2,
                              nc=2),
            out_type=jax.ShapeDtypeStruct((b * N, C2), jnp.float32),
            mesh=mesh,
            scratch_types=[
                pltpu.VMEM((3 * qpw,), jnp.int32),
                pltpu.VMEM((2, 3 * CHUNK, 16), jnp.float32),
                pltpu.VMEM((2, 3 * CHUNK, C2), jnp.float32),
                pltpu.VMEM((2, CHUNK, C2), jnp.float32),
                pltpu.SemaphoreType.DMA,
                pltpu.SemaphoreType.DMA,
                pltpu.SemaphoreType.DMA,
                pltpu.SemaphoreType.DMA,
            ],
        )(table, gidx, wrep)
        return interp.reshape(b, N, C2)

    def mlp_stage(p1, interp):
        b = p1.shape[0]
        return pl.pallas_call(
            _mlp_body,
            grid=(b, N // TN),
            in_specs=[
                pl.BlockSpec((1, C1, TN), lambda b, n: (b, 0, n)),
                pl.BlockSpec((1, TN, C2), lambda b, n: (b, n, 0)),
                pl.BlockSpec((O1, C1), lambda b, n: (0, 0)),
                pl.BlockSpec((O1, C2), lambda b, n: (0, 0)),
                pl.BlockSpec((O1, 1), lambda b, n: (0, 0)),
                pl.BlockSpec((O2, O1), lambda b, n: (0, 0)),
                pl.BlockSpec((O2, 1), lambda b, n: (0, 0)),
            ],
            out_specs=pl.BlockSpec((1, O2, TN), lambda b, n: (b, 0, n)),
            out_shape=jax.ShapeDtypeStruct((b, O2, N), jnp.float32),
            compiler_params=pltpu.CompilerParams(
                dimension_semantics=("parallel", "arbitrary"),
            ),
        )(p1, interp, w1a, w1b, b1c, W2, b2c)

    ngroups = 4
    half = B // ngroups
    outs = []
    tops = []
    for g in range(ngroups):
        sl = slice(g * half, (g + 1) * half)
        tops.append((top3_stage(xyz1[sl], xyz2[sl]), sl))
    for g in range(ngroups):
        (gidx4, wrep4), sl = tops[g]
        table = jnp.transpose(points2[sl], (0, 2, 1)).reshape(half * S, C2)
        interp = sc_stage(table, gidx4, wrep4)
        outs.append(mlp_stage(points1[sl], interp))
    return jnp.concatenate(outs, axis=0)
